# Initial kernel scaffold; baseline (speedup 1.0000x reference)
import functools
import jax
import jax.numpy as jnp
from jax import lax
from jax.experimental import pallas as pl
from jax.experimental.pallas import tpu as pltpu
from jax.experimental.pallas import tpu_sc as plsc

N = 10000
E = 320000
NCORE = 2
NSUB = 16
NT = NCORE * NSUB
EPT = E // NT
CH = 80
NCH = EPT // CH
D = 64

_mesh = plsc.VectorSubcoreMesh(
    core_axis_name="c", subcore_axis_name="s", num_cores=NCORE, num_subcores=NSUB
)


@functools.partial(
    pl.kernel,
    out_type=(
        jax.ShapeDtypeStruct((NCORE, N, D), jnp.float32),
        jax.ShapeDtypeStruct((NCORE, N), jnp.float32),
    ),
    mesh=_mesh,
    scratch_types=[
        pltpu.VMEM((NCH, CH), jnp.int32),
        pltpu.VMEM((NCH, CH), jnp.int32),
        pltpu.VMEM((NCH, CH), jnp.float32),
        pltpu.VMEM((NCH, CH), jnp.float32),
        pltpu.VMEM((N,), jnp.float32),
        pltpu.VMEM((N,), jnp.float32),
        pltpu.VMEM((CH, D), jnp.float32),
        pltpu.VMEM((125, D), jnp.float32),
        pltpu.VMEM((2000,), jnp.float32),
        pltpu.VMEM_SHARED((N, D), jnp.float32),
        pltpu.VMEM_SHARED((N,), jnp.float32),
        pltpu.SemaphoreType.DMA,
    ],
)
def _sc_layer64(src_hbm, dst_hbm, e_hbm, s_hbm, d_hbm, h_hbm,
                num_out, den_out,
                src_v, dst_v, e_v, ex_v, s_v, d_v, row_v, zrow_v, zden_v,
                num_sh, den_sh, sem):
    cid = lax.axis_index("c")
    sid = lax.axis_index("s")
    wid = cid * NSUB + sid

    pltpu.sync_copy(src_hbm.at[wid], src_v)
    pltpu.sync_copy(dst_hbm.at[wid], dst_v)
    pltpu.sync_copy(e_hbm.at[wid], e_v)
    pltpu.sync_copy(s_hbm, s_v)
    pltpu.sync_copy(d_hbm, d_v)

    zero16 = jnp.zeros((16,), jnp.float32)

    def _zfill(r, carry):
        for k in range(4):
            zrow_v[r, pl.ds(k * 16, 16)] = zero16
        return carry

    lax.fori_loop(0, 125, _zfill, 0)

    def _zfill1(r, carry):
        zden_v[pl.ds(r * 16, 16)] = zero16
        return carry

    lax.fori_loop(0, 125, _zfill1, 0)

    for q in range(5):
        pltpu.sync_copy(zrow_v, num_sh.at[pl.ds(sid * 625 + q * 125, 125)])

    @pl.when(sid < 5)
    def _():
        pltpu.sync_copy(zden_v, den_sh.at[pl.ds(sid * 2000, 2000)])

    plsc.subcore_barrier()

    def _p1(c, carry):
        for j in range(CH // 16):
            sl = pl.ds(j * 16, 16)
            srcv = src_v[c, sl]
            dstv = dst_v[c, sl]
            sg = plsc.load_gather(s_v, [srcv])
            dg = plsc.load_gather(d_v, [dstv])
            a = sg + dg + e_v[c, sl]
            a = jnp.where(a >= 0.0, a, a * jnp.float32(0.2))
            ex_v[c, sl] = jnp.exp(a)
        pltpu.sync_copy(ex_v.at[c], den_sh.at[dst_v.at[c]], add=True)
        return carry

    lax.fori_loop(0, NCH, _p1, 0)

    def _p2(c, carry):
        pltpu.async_copy(h_hbm.at[src_v.at[c]], row_v, sem).wait()

        def _scale(r, inner):
            f = ex_v[c, r]
            fb = jnp.full((16,), f, jnp.float32)
            for k in range(D // 16):
                sl = pl.ds(k * 16, 16)
                row_v[r, sl] = row_v[r, sl] * fb
            return inner

        lax.fori_loop(0, CH, _scale, 0)
        pltpu.sync_copy(row_v, num_sh.at[dst_v.at[c]], add=True)
        return carry

    lax.fori_loop(0, NCH, _p2, 0)

    plsc.subcore_barrier()

    pltpu.sync_copy(num_sh.at[pl.ds(sid * 625, 625)],
                    num_out.at[cid, pl.ds(sid * 625, 625)])

    @pl.when(sid < 5)
    def _():
        pltpu.sync_copy(den_sh.at[pl.ds(sid * 2000, 2000)],
                        den_out.at[cid, pl.ds(sid * 2000, 2000)])


@functools.partial(
    pl.kernel,
    out_type=(
        jax.ShapeDtypeStruct((NCORE, N), jnp.float32),
        jax.ShapeDtypeStruct((NCORE, N), jnp.float32),
    ),
    mesh=_mesh,
    scratch_types=[
        pltpu.VMEM((NCH, CH), jnp.int32),
        pltpu.VMEM((NCH, CH), jnp.int32),
        pltpu.VMEM((NCH, CH), jnp.float32),
        pltpu.VMEM((NCH, CH), jnp.float32),
        pltpu.VMEM((NCH, CH), jnp.float32),
        pltpu.VMEM((N,), jnp.float32),
        pltpu.VMEM((N,), jnp.float32),
        pltpu.VMEM((N,), jnp.float32),
        pltpu.VMEM((2000,), jnp.float32),
        pltpu.VMEM_SHARED((N,), jnp.float32),
        pltpu.VMEM_SHARED((N,), jnp.float32),
    ],
)
def _sc_layer1(src_hbm, dst_hbm, e_hbm, s_hbm, d_hbm, h_hbm,
               num_out, den_out,
               src_v, dst_v, e_v, ex_v, cb_v, s_v, d_v, h_v, zden_v,
               num_sh, den_sh):
    cid = lax.axis_index("c")
    sid = lax.axis_index("s")
    wid = cid * NSUB + sid

    pltpu.sync_copy(src_hbm.at[wid], src_v)
    pltpu.sync_copy(dst_hbm.at[wid], dst_v)
    pltpu.sync_copy(e_hbm.at[wid], e_v)
    pltpu.sync_copy(s_hbm, s_v)
    pltpu.sync_copy(d_hbm, d_v)
    pltpu.sync_copy(h_hbm, h_v)

    zero16 = jnp.zeros((16,), jnp.float32)

    def _zfill1(r, carry):
        zden_v[pl.ds(r * 16, 16)] = zero16
        return carry

    lax.fori_loop(0, 125, _zfill1, 0)

    @pl.when(sid < 5)
    def _():
        pltpu.sync_copy(zden_v, num_sh.at[pl.ds(sid * 2000, 2000)])
        pltpu.sync_copy(zden_v, den_sh.at[pl.ds(sid * 2000, 2000)])

    plsc.subcore_barrier()

    def _p1(c, carry):
        for j in range(CH // 16):
            sl = pl.ds(j * 16, 16)
            srcv = src_v[c, sl]
            dstv = dst_v[c, sl]
            sg = plsc.load_gather(s_v, [srcv])
            dg = plsc.load_gather(d_v, [dstv])
            a = sg + dg + e_v[c, sl]
            a = jnp.where(a >= 0.0, a, a * jnp.float32(0.2))
            ex = jnp.exp(a)
            ex_v[c, sl] = ex
            hv = plsc.load_gather(h_v, [srcv])
            cb_v[c, sl] = ex * hv
        pltpu.sync_copy(ex_v.at[c], den_sh.at[dst_v.at[c]], add=True)
        pltpu.sync_copy(cb_v.at[c], num_sh.at[dst_v.at[c]], add=True)
        return carry

    lax.fori_loop(0, NCH, _p1, 0)

    plsc.subcore_barrier()

    @pl.when(sid < 5)
    def _():
        pltpu.sync_copy(num_sh.at[pl.ds(sid * 2000, 2000)],
                        num_out.at[cid, pl.ds(sid * 2000, 2000)])
        pltpu.sync_copy(den_sh.at[pl.ds(sid * 2000, 2000)],
                        den_out.at[cid, pl.ds(sid * 2000, 2000)])


def _edge_body(ea_ref, we1_ref, ae1_ref, we2_ref, ae2_ref, we3_ref, ae3_ref, out_ref):
    v1 = jnp.dot(we1_ref[...], ae1_ref[...], preferred_element_type=jnp.float32)
    v2 = jnp.dot(we2_ref[...], ae2_ref[...], preferred_element_type=jnp.float32)
    v3 = jnp.dot(we3_ref[...], ae3_ref[...], preferred_element_type=jnp.float32)
    ve = jnp.concatenate([v1, v2, v3, jnp.zeros((16, 5), jnp.float32)], axis=1)
    out_ref[...] = jnp.dot(ea_ref[...], ve, preferred_element_type=jnp.float32)


def _edge_tc(edge_attr, We1, ae1, We2, ae2, We3, ae3):
    blk = 8000
    grid = E // blk
    full = lambda shape: pl.BlockSpec(shape, lambda i: (0, 0))
    return pl.pallas_call(
        _edge_body,
        grid=(grid,),
        in_specs=[
            pl.BlockSpec((blk, 16), lambda i: (i, 0)),
            full(We1.shape), full(ae1.shape),
            full(We2.shape), full(ae2.shape),
            full(We3.shape), full(ae3.shape),
        ],
        out_specs=pl.BlockSpec((blk, 8), lambda i: (i, 0)),
        out_shape=jax.ShapeDtypeStruct((E, 8), jnp.float32),
    )(edge_attr, We1, ae1, We2, ae2, We3, ae3)


def _dense0_body(x_ref, w_ref, as_ref, h_ref, sd_ref):
    h = jnp.dot(x_ref[...], w_ref[...], preferred_element_type=jnp.float32)
    h_ref[...] = h
    sd_ref[...] = jnp.dot(h, as_ref[...], preferred_element_type=jnp.float32)


def _dense0(x, W, As):
    return pl.pallas_call(
        _dense0_body,
        out_shape=(
            jax.ShapeDtypeStruct((N, W.shape[1]), jnp.float32),
            jax.ShapeDtypeStruct((N, 8), jnp.float32),
        ),
    )(x, W, As)


def _layer_body(n0_ref, n1_ref, d0_ref, d1_ref, b_ref, w_ref, as_ref, h_ref, sd_ref):
    den = d0_ref[...] + d1_ref[...] + jnp.float32(1e-16)
    xn = (n0_ref[...] + n1_ref[...]) / den + b_ref[...]
    xn = jnp.maximum(xn, 0.0)
    h = jnp.dot(xn, w_ref[...], preferred_element_type=jnp.float32)
    h_ref[...] = h
    sd_ref[...] = jnp.dot(h, as_ref[...], preferred_element_type=jnp.float32)


def _tc_layer(n0, n1, d0, d1, b, W, As):
    return pl.pallas_call(
        _layer_body,
        out_shape=(
            jax.ShapeDtypeStruct((N, W.shape[1]), jnp.float32),
            jax.ShapeDtypeStruct((N, 8), jnp.float32),
        ),
    )(n0, n1, d0, d1, b, W, As)


def _final_body(n0_ref, n1_ref, d0_ref, d1_ref, b_ref, out_ref):
    den = d0_ref[...] + d1_ref[...] + jnp.float32(1e-16)
    out_ref[...] = (n0_ref[...] + n1_ref[...]) / den + b_ref[...]


def _tc_final(n0, n1, d0, d1, b):
    return pl.pallas_call(
        _final_body,
        out_shape=jax.ShapeDtypeStruct((N, 1), jnp.float32),
    )(n0, n1, d0, d1, b)


def kernel(x, edge_index, edge_attr,
           W1, att_src1, att_dst1, We1, att_e1, b1,
           W2, att_src2, att_dst2, We2, att_e2, b2,
           W3, att_src3, att_dst3, We3, att_e3, b3):
    src = edge_index[0].reshape(NT, NCH, CH)
    dst = edge_index[1].reshape(NT, NCH, CH)

    e_all = _edge_tc(edge_attr,
                     We1, att_e1.reshape(-1, 1),
                     We2, att_e2.reshape(-1, 1),
                     We3, att_e3.reshape(-1, 1))
    e1 = e_all[:, 0].reshape(NT, NCH, CH)
    e2 = e_all[:, 1].reshape(NT, NCH, CH)
    e3 = e_all[:, 2].reshape(NT, NCH, CH)

    As1 = jnp.zeros((D, 8), jnp.float32).at[:, 0].set(att_src1).at[:, 1].set(att_dst1)
    As2 = jnp.zeros((D, 8), jnp.float32).at[:, 0].set(att_src2).at[:, 1].set(att_dst2)
    As3 = jnp.zeros((8, 8), jnp.float32).at[0, 0].set(att_src3[0]).at[0, 1].set(att_dst3[0])
    W3p = jnp.zeros((D, 8), jnp.float32).at[:, 0].set(W3[:, 0])

    h1, sd1 = _dense0(x, W1, As1)
    num1, den1 = _sc_layer64(src, dst, e1, sd1[:, 0], sd1[:, 1], h1)

    h2, sd2 = _tc_layer(num1[0], num1[1], den1[0][:, None], den1[1][:, None],
                        b1[None, :], W2, As2)
    num2, den2 = _sc_layer64(src, dst, e2, sd2[:, 0], sd2[:, 1], h2)

    h3p, sd3 = _tc_layer(num2[0], num2[1], den2[0][:, None], den2[1][:, None],
                         b2[None, :], W3p, As3)
    num3, den3 = _sc_layer1(src, dst, e3, sd3[:, 0], sd3[:, 1], h3p[:, 0])

    return _tc_final(num3[0][:, None], num3[1][:, None],
                     den3[0][:, None], den3[1][:, None], b3.reshape(1, 1))


# speedup vs baseline: 50.3135x; 50.3135x over previous
import functools
from jax.lax import Precision as _Prec
import jax
import jax.numpy as jnp
from jax import lax
from jax.experimental import pallas as pl
from jax.experimental.pallas import tpu as pltpu
from jax.experimental.pallas import tpu_sc as plsc

N = 10000
E = 320000
NCORE = 2
NSUB = 16
NT = NCORE * NSUB
EPT = E // NT
CH = 80
NCH = EPT // CH
D = 64

_mesh = plsc.VectorSubcoreMesh(
    core_axis_name="c", subcore_axis_name="s", num_cores=NCORE, num_subcores=NSUB
)


@functools.partial(
    pl.kernel,
    out_type=(
        jax.ShapeDtypeStruct((NCORE, N, D), jnp.float32),
        jax.ShapeDtypeStruct((NCORE * N,), jnp.float32),
    ),
    mesh=_mesh,
    compiler_params=pltpu.CompilerParams(needs_layout_passes=False, use_tc_tiling_on_sc=False),
    scratch_types=[
        pltpu.VMEM((NCH, CH), jnp.int32),
        pltpu.VMEM((NCH, CH), jnp.int32),
        pltpu.VMEM((NCH, CH), jnp.float32),
        pltpu.VMEM((NCH, CH), jnp.float32),
        pltpu.VMEM((N,), jnp.float32),
        pltpu.VMEM((N,), jnp.float32),
        pltpu.VMEM((CH, D), jnp.float32),
        pltpu.VMEM((CH, D), jnp.float32),
        pltpu.VMEM((CH, D), jnp.float32),
        pltpu.VMEM((CH, D), jnp.float32),
        pltpu.VMEM((104, D), jnp.float32),
        pltpu.VMEM((2000,), jnp.float32),
        pltpu.VMEM_SHARED((N, D), jnp.float32),
        pltpu.VMEM_SHARED((N,), jnp.float32),
        pltpu.SemaphoreType.DMA,
        pltpu.SemaphoreType.DMA,
        pltpu.SemaphoreType.DMA,
        pltpu.SemaphoreType.DMA,
        pltpu.SemaphoreType.DMA,
    ],
)
def _sc_layer64(src_hbm, dst_hbm, e_hbm, s_hbm, d_hbm, h_hbm,
                num_out, den_out,
                src_v, dst_v, e_v, ex_v, s_v, d_v, gb_a, gb_b, sb_a, sb_b,
                zrow_v, zden_v, num_sh, den_sh, sem_den, sg_a, sg_b, ss_a, ss_b):
    cid = lax.axis_index("c")
    sid = lax.axis_index("s")
    wid = cid * NSUB + sid

    pltpu.sync_copy(src_hbm.at[wid], src_v)
    pltpu.sync_copy(dst_hbm.at[wid], dst_v)
    pltpu.sync_copy(e_hbm.at[wid], e_v)
    pltpu.sync_copy(s_hbm, s_v)
    pltpu.sync_copy(d_hbm, d_v)

    zero16 = jnp.zeros((16,), jnp.float32)

    def _zfill(r, carry):
        for k in range(4):
            zrow_v[r, pl.ds(k * 16, 16)] = zero16
        return carry

    lax.fori_loop(0, 104, _zfill, 0)

    def _zfill1(r, carry):
        zden_v[pl.ds(r * 16, 16)] = zero16
        return carry

    lax.fori_loop(0, 125, _zfill1, 0)

    def _zclr(q, carry):
        pltpu.sync_copy(zrow_v, num_sh.at[pl.ds(sid * 624 + q * 104, 104)])
        return carry

    lax.fori_loop(0, 6, _zclr, 0)

    @pl.when(sid == NSUB - 1)
    def _():
        pltpu.sync_copy(zrow_v.at[pl.ds(0, 16)], num_sh.at[pl.ds(9984, 16)])

    @pl.when(sid < 5)
    def _():
        pltpu.sync_copy(zden_v, den_sh.at[pl.ds(sid * 2000, 2000)])

    plsc.subcore_barrier()

    def _p1(c, carry):
        for j in range(CH // 16):
            sl = pl.ds(j * 16, 16)
            srcv = src_v[c, sl]
            dstv = dst_v[c, sl]
            sg = plsc.load_gather(s_v, [srcv])
            dg = plsc.load_gather(d_v, [dstv])
            a = sg + dg + e_v[c, sl]
            a = jnp.where(a >= 0.0, a, a * jnp.float32(0.2))
            ex_v[c, sl] = jnp.exp(a)
        pltpu.async_copy(ex_v.at[c], den_sh.at[dst_v.at[c]], sem_den, add=True)
        return carry

    lax.fori_loop(0, NCH, _p1, 0)

    def _gissue(c, gb, sg):
        pltpu.async_copy(h_hbm.at[src_v.at[c]], gb, sg)

    def _gwait(c, gb, sg):
        pltpu.make_async_copy(h_hbm.at[src_v.at[c]], gb, sg).wait()

    def _sissue(c, sb, ss):
        pltpu.async_copy(sb, num_sh.at[dst_v.at[c]], ss, add=True)

    def _swait(c, sb, ss):
        pltpu.make_async_copy(sb, num_sh.at[dst_v.at[c]], ss).wait()

    def _scale(c, gb, sb):
        def _sg(g, inner):
            exv = ex_v[c, pl.ds(g * 16, 16)]
            base = g * 16
            for i in range(16):
                fb = jnp.full((16,), exv[i], jnp.float32)
                for k in range(D // 16):
                    sl = pl.ds(k * 16, 16)
                    sb[base + i, sl] = gb[base + i, sl] * fb
            return inner

        lax.fori_loop(0, CH // 16, _sg, 0)

    def _chunk(c, gb, sg, sb, ss, first, last):
        _gwait(c, gb, sg)
        if not first:
            _swait(c - 2, sb, ss)
        _scale(c, gb, sb)
        if not last:
            _gissue(c + 2, gb, sg)
        _sissue(c, sb, ss)

    _gissue(0, gb_a, sg_a)
    _gissue(1, gb_b, sg_b)

    _chunk(0, gb_a, sg_a, sb_a, ss_a, True, False)
    _chunk(1, gb_b, sg_b, sb_b, ss_b, True, False)

    def _pair(i, carry):
        c0 = 2 * i
        _chunk(c0, gb_a, sg_a, sb_a, ss_a, False, False)
        _chunk(c0 + 1, gb_b, sg_b, sb_b, ss_b, False, False)
        return carry

    lax.fori_loop(1, 61, _pair, 0)
    _chunk(122, gb_a, sg_a, sb_a, ss_a, False, False)
    _chunk(123, gb_b, sg_b, sb_b, ss_b, False, True)
    _chunk(124, gb_a, sg_a, sb_a, ss_a, False, True)

    _swait(NCH - 2, sb_b, ss_b)
    _swait(NCH - 1, sb_a, ss_a)

    def _dend(c, carry):
        pltpu.make_async_copy(ex_v.at[c], den_sh.at[dst_v.at[c]], sem_den).wait()
        return carry

    lax.fori_loop(0, NCH, _dend, 0)

    plsc.subcore_barrier()

    def _nout(q, carry):
        pltpu.sync_copy(num_sh.at[pl.ds(sid * 624 + q * 104, 104)],
                        num_out.at[cid, pl.ds(sid * 624 + q * 104, 104)])
        return carry

    lax.fori_loop(0, 6, _nout, 0)

    @pl.when(sid == NSUB - 1)
    def _():
        pltpu.sync_copy(num_sh.at[pl.ds(9984, 16)],
                        num_out.at[cid, pl.ds(9984, 16)])

    @pl.when(sid < 5)
    def _():
        pltpu.sync_copy(den_sh.at[pl.ds(sid * 2000, 2000)],
                        den_out.at[pl.ds(cid * N + sid * 2000, 2000)])


@functools.partial(
    pl.kernel,
    out_type=(
        jax.ShapeDtypeStruct((NCORE * N,), jnp.float32),
        jax.ShapeDtypeStruct((NCORE * N,), jnp.float32),
    ),
    mesh=_mesh,
    compiler_params=pltpu.CompilerParams(needs_layout_passes=False, use_tc_tiling_on_sc=False),
    scratch_types=[
        pltpu.VMEM((NCH, CH), jnp.int32),
        pltpu.VMEM((NCH, CH), jnp.int32),
        pltpu.VMEM((NCH, CH), jnp.float32),
        pltpu.VMEM((NCH, CH), jnp.float32),
        pltpu.VMEM((NCH, CH), jnp.float32),
        pltpu.VMEM((N,), jnp.float32),
        pltpu.VMEM((N,), jnp.float32),
        pltpu.VMEM((N,), jnp.float32),
        pltpu.VMEM((2000,), jnp.float32),
        pltpu.VMEM_SHARED((N,), jnp.float32),
        pltpu.VMEM_SHARED((N,), jnp.float32),
        pltpu.SemaphoreType.DMA,
        pltpu.SemaphoreType.DMA,
    ],
)
def _sc_layer1(src_hbm, dst_hbm, e_hbm, s_hbm, d_hbm, h_hbm,
               num_out, den_out,
               src_v, dst_v, e_v, ex_v, cb_v, s_v, d_v, h_v, zden_v,
               num_sh, den_sh, sem_d, sem_n):
    cid = lax.axis_index("c")
    sid = lax.axis_index("s")
    wid = cid * NSUB + sid

    pltpu.sync_copy(src_hbm.at[wid], src_v)
    pltpu.sync_copy(dst_hbm.at[wid], dst_v)
    pltpu.sync_copy(e_hbm.at[wid], e_v)
    pltpu.sync_copy(s_hbm, s_v)
    pltpu.sync_copy(d_hbm, d_v)
    pltpu.sync_copy(h_hbm, h_v)

    zero16 = jnp.zeros((16,), jnp.float32)

    def _zfill1(r, carry):
        zden_v[pl.ds(r * 16, 16)] = zero16
        return carry

    lax.fori_loop(0, 125, _zfill1, 0)

    @pl.when(sid < 5)
    def _():
        pltpu.sync_copy(zden_v, num_sh.at[pl.ds(sid * 2000, 2000)])
        pltpu.sync_copy(zden_v, den_sh.at[pl.ds(sid * 2000, 2000)])

    plsc.subcore_barrier()

    def _p1(c, carry):
        for j in range(CH // 16):
            sl = pl.ds(j * 16, 16)
            srcv = src_v[c, sl]
            dstv = dst_v[c, sl]
            sg = plsc.load_gather(s_v, [srcv])
            dg = plsc.load_gather(d_v, [dstv])
            a = sg + dg + e_v[c, sl]
            a = jnp.where(a >= 0.0, a, a * jnp.float32(0.2))
            ex = jnp.exp(a)
            ex_v[c, sl] = ex
            hv = plsc.load_gather(h_v, [srcv])
            cb_v[c, sl] = ex * hv
        pltpu.async_copy(ex_v.at[c], den_sh.at[dst_v.at[c]], sem_d, add=True)
        pltpu.async_copy(cb_v.at[c], num_sh.at[dst_v.at[c]], sem_n, add=True)
        return carry

    lax.fori_loop(0, NCH, _p1, 0)

    def _drain(c, carry):
        pltpu.make_async_copy(ex_v.at[c], den_sh.at[dst_v.at[c]], sem_d).wait()
        pltpu.make_async_copy(cb_v.at[c], num_sh.at[dst_v.at[c]], sem_n).wait()
        return carry

    lax.fori_loop(0, NCH, _drain, 0)

    plsc.subcore_barrier()

    @pl.when(sid < 5)
    def _():
        pltpu.sync_copy(num_sh.at[pl.ds(sid * 2000, 2000)],
                        num_out.at[pl.ds(cid * N + sid * 2000, 2000)])
        pltpu.sync_copy(den_sh.at[pl.ds(sid * 2000, 2000)],
                        den_out.at[pl.ds(cid * N + sid * 2000, 2000)])


def _edge_body(ea_ref, bd_ref, out_ref):
    out_ref[...] = jnp.dot(ea_ref[...], bd_ref[...],
                           preferred_element_type=jnp.float32, precision=_Prec.HIGHEST)


def _edge_tc(ea_r, bd):
    return pl.pallas_call(
        _edge_body,
        out_shape=jax.ShapeDtypeStruct((ea_r.shape[0], 384), jnp.float32),
    )(ea_r, bd)


def _dense0_body(x_ref, w_ref, as_ref, we1_ref, ae1_ref, we2_ref, ae2_ref,
                 we3_ref, ae3_ref, h_ref, sd_ref, v_ref):
    h = jnp.dot(x_ref[...], w_ref[...], preferred_element_type=jnp.float32, precision=_Prec.HIGHEST)
    h_ref[...] = h
    sd_ref[...] = jnp.dot(h, as_ref[...], preferred_element_type=jnp.float32, precision=_Prec.HIGHEST)
    v1 = jnp.dot(we1_ref[...], ae1_ref[...], preferred_element_type=jnp.float32, precision=_Prec.HIGHEST)
    v2 = jnp.dot(we2_ref[...], ae2_ref[...], preferred_element_type=jnp.float32, precision=_Prec.HIGHEST)
    v3 = jnp.dot(we3_ref[...], ae3_ref[...], preferred_element_type=jnp.float32, precision=_Prec.HIGHEST)
    v_ref[...] = jnp.concatenate([v1, v2, v3, jnp.zeros((16, 5), jnp.float32)], axis=1)


def _dense0(x, W, As, We1, ae1, We2, ae2, We3, ae3):
    return pl.pallas_call(
        _dense0_body,
        out_shape=(
            jax.ShapeDtypeStruct((N, W.shape[1]), jnp.float32),
            jax.ShapeDtypeStruct((N, 8), jnp.float32),
            jax.ShapeDtypeStruct((16, 8), jnp.float32),
        ),
    )(x, W, As, We1, ae1, We2, ae2, We3, ae3)


def _layer_body(n0_ref, n1_ref, d0_ref, d1_ref, b_ref, w_ref, as_ref, h_ref, sd_ref):
    den = d0_ref[...] + d1_ref[...] + jnp.float32(1e-16)
    xn = (n0_ref[...] + n1_ref[...]) / den + b_ref[...]
    xn = jnp.maximum(xn, 0.0)
    h = jnp.dot(xn, w_ref[...], preferred_element_type=jnp.float32, precision=_Prec.HIGHEST)
    h_ref[...] = h
    sd_ref[...] = jnp.dot(h, as_ref[...], preferred_element_type=jnp.float32, precision=_Prec.HIGHEST)


def _tc_layer(n0, n1, d0, d1, b, W, As):
    return pl.pallas_call(
        _layer_body,
        out_shape=(
            jax.ShapeDtypeStruct((N, W.shape[1]), jnp.float32),
            jax.ShapeDtypeStruct((N, 8), jnp.float32),
        ),
    )(n0, n1, d0, d1, b, W, As)


def _final_body(n0_ref, n1_ref, d0_ref, d1_ref, b_ref, out_ref):
    den = d0_ref[...] + d1_ref[...] + jnp.float32(1e-16)
    out_ref[...] = (n0_ref[...] + n1_ref[...]) / den + b_ref[...]


def _tc_final(n0, n1, d0, d1, b):
    return pl.pallas_call(
        _final_body,
        out_shape=jax.ShapeDtypeStruct((N, 1), jnp.float32),
    )(n0, n1, d0, d1, b)


def kernel(x, edge_index, edge_attr,
           W1, att_src1, att_dst1, We1, att_e1, b1,
           W2, att_src2, att_dst2, We2, att_e2, b2,
           W3, att_src3, att_dst3, We3, att_e3, b3):
    src = edge_index[0].reshape(NT, NCH, CH)
    dst = edge_index[1].reshape(NT, NCH, CH)

    As1 = jnp.zeros((D, 8), jnp.float32).at[:, 0].set(att_src1).at[:, 1].set(att_dst1)
    As2 = jnp.zeros((D, 8), jnp.float32).at[:, 0].set(att_src2).at[:, 1].set(att_dst2)
    As3 = jnp.zeros((8, 8), jnp.float32).at[0, 0].set(att_src3[0]).at[0, 1].set(att_dst3[0])
    W3p = jnp.zeros((D, 8), jnp.float32).at[:, 0].set(W3[:, 0])

    h1, sd1, v_all = _dense0(x, W1, As1,
                             We1, att_e1.reshape(-1, 1),
                             We2, att_e2.reshape(-1, 1),
                             We3, att_e3.reshape(-1, 1))

    ea_r = edge_attr.reshape(E // 128, 128 * 16)
    bd = (jnp.eye(128, dtype=jnp.float32)[:, None, None, :]
          * v_all[None, :, :3, None]).reshape(2048, 384)
    e_mat = _edge_tc(ea_r, bd)
    e1 = e_mat[:, 0:128].reshape(NT, NCH, CH)
    e2 = e_mat[:, 128:256].reshape(NT, NCH, CH)
    e3 = e_mat[:, 256:384].reshape(NT, NCH, CH)
    num1, den1 = _sc_layer64(src, dst, e1, sd1[:, 0], sd1[:, 1], h1)

    h2, sd2 = _tc_layer(num1[0], num1[1], den1[:N][:, None], den1[N:][:, None],
                        b1[None, :], W2, As2)
    num2, den2 = _sc_layer64(src, dst, e2, sd2[:, 0], sd2[:, 1], h2)

    h3p, sd3 = _tc_layer(num2[0], num2[1], den2[:N][:, None], den2[N:][:, None],
                         b2[None, :], W3p, As3)
    num3, den3 = _sc_layer1(src, dst, e3, sd3[:, 0], sd3[:, 1], h3p[:, 0])

    return _tc_final(num3[:N][:, None], num3[N:][:, None],
                     den3[:N][:, None], den3[N:][:, None], b3.reshape(1, 1))
